# Initial kernel scaffold; baseline (speedup 1.0000x reference)
import jax
import jax.numpy as jnp
from jax.experimental import pallas as pl


def kernel(x, edge_index, W1a, b1a, g1a, be1a, W2a, b2a, W1b, b1b, g1b, be1b, W2b, b2b, Wl1, bl1, Wl2, bl2):
    raise NotImplementedError("write your pallas kernel here")


# speedup vs baseline: 6.3967x; 6.3967x over previous
import functools

import jax
import jax.numpy as jnp
from jax import lax
from jax.experimental import pallas as pl
from jax.experimental.pallas import tpu as pltpu
from jax.experimental.pallas import tpu_sc as plsc

_NC = 2
_NS = 16


def _sc_segment_sum(x, edge_index):
  n, d = x.shape
  e = edge_index.shape[1]
  ch = 128
  assert e % ch == 0, e
  nchunks = e // ch
  nw = _NC * _NS
  tmax = (nchunks + nw - 1) // nw
  rch = 128
  npad = -(-n // (_NS * rch)) * (_NS * rch)
  rpt = npad // _NS
  nr = rpt // rch
  assert d % 16 == 0, d

  mesh = plsc.VectorSubcoreMesh(core_axis_name="c", subcore_axis_name="s")

  @functools.partial(
      pl.kernel,
      out_type=jax.ShapeDtypeStruct((_NC, npad, d), jnp.float32),
      mesh=mesh,
      scratch_types=[
          pltpu.VMEM((ch,), jnp.int32),
          pltpu.VMEM((ch,), jnp.int32),
          pltpu.VMEM((ch, d), jnp.float32),
          pltpu.VMEM((rch, d), jnp.float32),
          pltpu.VMEM_SHARED((npad, d), jnp.float32),
          pltpu.SemaphoreType.DMA,
      ],
  )
  def k(x_hbm, ei_hbm, out_hbm, sidx, didx, rows, zb, acc, sem):
    cid = lax.axis_index("c")
    sid = lax.axis_index("s")
    wid = sid * _NC + cid

    def zrow(i, carry):
      for j in range(d // 16):
        zb[i, pl.ds(j * 16, 16)] = jnp.zeros((16,), jnp.float32)
      return carry
    lax.fori_loop(0, rch, zrow, 0)
    r0 = sid * rpt
    for kk in range(nr):
      pltpu.sync_copy(zb, acc.at[pl.ds(r0 + kk * rch, rch)])
    plsc.subcore_barrier()

    def body(t, carry):
      cnum = wid + nw * t

      @pl.when(cnum < nchunks)
      def _():
        base = cnum * ch
        pltpu.sync_copy(ei_hbm.at[0, pl.ds(base, ch)], sidx)
        pltpu.sync_copy(ei_hbm.at[1, pl.ds(base, ch)], didx)
        pltpu.async_copy(x_hbm.at[sidx], rows, sem).wait()
        pltpu.sync_copy(rows, acc.at[didx], add=True)

      return carry
    lax.fori_loop(0, tmax, body, 0)
    plsc.subcore_barrier()

    for kk in range(nr):
      pltpu.sync_copy(acc.at[pl.ds(r0 + kk * rch, rch)], zb)
      pltpu.sync_copy(zb, out_hbm.at[cid, pl.ds(r0 + kk * rch, rch)])

  return k(x, edge_index)[:, :n, :]


def _matmul_t(h, w):
  return lax.dot_general(h, w, (((1,), (1,)), ((), ())),
                         preferred_element_type=jnp.float32)


def _gin_mlp(h, w1, b1, g, be, w2, b2):
  h = _matmul_t(h, w1) + b1
  m = jnp.mean(h, axis=0, keepdims=True)
  v = jnp.mean((h - m) * (h - m), axis=0, keepdims=True)
  h = (h - m) * lax.rsqrt(v + 1e-5) * g + be
  h = jnp.maximum(h, 0.0)
  h = _matmul_t(h, w2) + b2
  return jnp.maximum(h, 0.0)


def _tc_layer_a(x, p, w1, b1, g, be, w2, b2):
  n, d = x.shape

  def body(x_ref, p_ref, w1_ref, b1_ref, g_ref, be_ref, w2_ref, b2_ref, o_ref):
    h = x_ref[...] + p_ref[0, :, :] + p_ref[1, :, :]
    o_ref[...] = _gin_mlp(h, w1_ref[...], b1_ref[...], g_ref[...],
                          be_ref[...], w2_ref[...], b2_ref[...])

  return pl.pallas_call(
      body, out_shape=jax.ShapeDtypeStruct((n, d), jnp.float32),
  )(x, p, w1, b1.reshape(1, -1), g.reshape(1, -1), be.reshape(1, -1),
    w2, b2.reshape(1, -1))


def _tc_layer_b(x, p, w1, b1, g, be, w2, b2, wl1, bl1, wl2, bl2):
  n, d = x.shape
  dout = wl2.shape[0]

  def body(x_ref, p_ref, w1_ref, b1_ref, g_ref, be_ref, w2_ref, b2_ref,
           wl1_ref, bl1_ref, wl2_ref, bl2_ref, o_ref):
    h = x_ref[...] + p_ref[0, :, :] + p_ref[1, :, :]
    h = _gin_mlp(h, w1_ref[...], b1_ref[...], g_ref[...], be_ref[...],
                 w2_ref[...], b2_ref[...])
    h = jnp.maximum(_matmul_t(h, wl1_ref[...]) + bl1_ref[...], 0.0)
    o_ref[...] = _matmul_t(h, wl2_ref[...]) + bl2_ref[...]

  return pl.pallas_call(
      body, out_shape=jax.ShapeDtypeStruct((n, dout), jnp.float32),
  )(x, p, w1, b1.reshape(1, -1), g.reshape(1, -1), be.reshape(1, -1),
    w2, b2.reshape(1, -1), wl1, bl1.reshape(1, -1), wl2, bl2.reshape(1, -1))


def kernel(x, edge_index, W1a, b1a, g1a, be1a, W2a, b2a,
           W1b, b1b, g1b, be1b, W2b, b2b, Wl1, bl1, Wl2, bl2):
  p = _sc_segment_sum(x, edge_index)
  h = _tc_layer_a(x, p, W1a, b1a, g1a, be1a, W2a, b2a)
  q = _sc_segment_sum(h, edge_index)
  return _tc_layer_b(h, q, W1b, b1b, g1b, be1b, W2b, b2b, Wl1, bl1, Wl2, bl2)
